# Initial kernel scaffold; baseline (speedup 1.0000x reference)
import jax
import jax.numpy as jnp
from jax.experimental import pallas as pl


def kernel(inputs, edge_index, W1, b1, p_pool, W_ih, W_hh, b_ih, b_hh, init_w, W2, b2):
    raise NotImplementedError("write your pallas kernel here")


# speedup vs baseline: 24.4441x; 24.4441x over previous
import functools

import jax
import jax.numpy as jnp
from jax import lax
from jax.experimental import pallas as pl
from jax.experimental.pallas import tpu as pltpu
from jax.experimental.pallas import tpu_sc as plsc

F32 = jnp.float32

_NC = 2
_NS = 16
_NW = _NC * _NS

_ROW_BLK = 2000
_C = 80


def _mesh():
    return plsc.VectorSubcoreMesh(core_axis_name="c", subcore_axis_name="s",
                                  num_cores=_NC, num_subcores=_NS)


def _sc_deg(dst3, zeros_n):
    nw, nch, c = dst3.shape
    n = zeros_n.shape[0]

    @functools.partial(
        pl.kernel, mesh=_mesh(),
        out_type=jax.ShapeDtypeStruct((_NC, n), F32),
        scratch_types=[pltpu.VMEM((nch, c), jnp.int32),
                       pltpu.VMEM((c,), F32),
                       pltpu.VMEM_SHARED((n,), F32)],
    )
    def k(dst_hbm, z_hbm, out_hbm, idx_v, ones_v, acc_sh):
        cid = lax.axis_index("c")
        sid = lax.axis_index("s")
        wid = sid * _NC + cid

        @pl.when(sid == 0)
        def _():
            pltpu.sync_copy(z_hbm, acc_sh)

        pltpu.sync_copy(dst_hbm.at[wid], idx_v)
        for j in range(c // 16):
            ones_v[pl.ds(16 * j, 16)] = jnp.ones((16,), F32)
        plsc.subcore_barrier()

        def chunk(i, carry):
            pltpu.sync_copy(ones_v, acc_sh.at[idx_v.at[i]], add=True)
            return carry
        lax.fori_loop(0, nch, chunk, 0)

        plsc.subcore_barrier()

        @pl.when(sid == 0)
        def _():
            pltpu.sync_copy(acc_sh, out_hbm.at[cid])

    return k(dst3, zeros_n)


def _sc_agg(y, src3, dst3, zeros_nd):
    nw, nch, c = src3.shape
    n, d = y.shape

    @functools.partial(
        pl.kernel, mesh=_mesh(),
        out_type=jax.ShapeDtypeStruct((_NC, n, d), F32),
        scratch_types=[pltpu.VMEM((nch, c), jnp.int32),
                       pltpu.VMEM((nch, c), jnp.int32),
                       pltpu.VMEM((c, d), F32),
                       pltpu.VMEM_SHARED((n, d), F32)],
    )
    def k(y_hbm, src_hbm, dst_hbm, z_hbm, out_hbm, si_v, di_v, rows_v, acc_sh):
        cid = lax.axis_index("c")
        sid = lax.axis_index("s")
        wid = sid * _NC + cid

        @pl.when(sid == 0)
        def _():
            pltpu.sync_copy(z_hbm, acc_sh)

        pltpu.sync_copy(src_hbm.at[wid], si_v)
        pltpu.sync_copy(dst_hbm.at[wid], di_v)
        plsc.subcore_barrier()

        def chunk(i, carry):
            pltpu.sync_copy(y_hbm.at[si_v.at[i]], rows_v)
            pltpu.sync_copy(rows_v, acc_sh.at[di_v.at[i]], add=True)
            return carry
        lax.fori_loop(0, nch, chunk, 0)

        plsc.subcore_barrier()

        @pl.when(sid == 0)
        def _():
            pltpu.sync_copy(acc_sh, out_hbm.at[cid])

    return k(y, src3, dst3, zeros_nd)


def _fc1_body(inp_ref, w1_ref, b1_ref, pp_ref, x_ref, s_ref):
    x = jnp.maximum(
        jnp.dot(inp_ref[...], w1_ref[...], preferred_element_type=F32) + b1_ref[...], 0.0)
    x_ref[...] = x
    pp = pp_ref[...]
    nrm = jnp.sqrt(jnp.sum(pp * pp)) + 1e-16
    s_ref[...] = jnp.tanh(
        jnp.dot(x, pp, preferred_element_type=F32) / nrm)


def _fc1(inputs, W1, b1r, ppc):
    n, d = inputs.shape
    return pl.pallas_call(
        _fc1_body,
        grid=(n // _ROW_BLK,),
        in_specs=[pl.BlockSpec((_ROW_BLK, d), lambda i: (i, 0)),
                  pl.BlockSpec((d, d), lambda i: (0, 0)),
                  pl.BlockSpec((1, d), lambda i: (0, 0)),
                  pl.BlockSpec((d, 1), lambda i: (0, 0))],
        out_specs=[pl.BlockSpec((_ROW_BLK, d), lambda i: (i, 0)),
                   pl.BlockSpec((_ROW_BLK, 1), lambda i: (i, 0))],
        out_shape=[jax.ShapeDtypeStruct((n, d), F32),
                   jax.ShapeDtypeStruct((n, 1), F32)],
    )(inputs, W1, b1r, ppc)


def _gru_body(perm_ref, vals_ref, x_ref, h0_ref, wih_ref, whh_ref,
              bih_ref, bhh_ref, w_ref, xt_ref):
    kk, d = xt_ref.shape

    def loop(i, carry):
        row = x_ref[pl.ds(perm_ref[i], 1), :]
        xt_ref[pl.ds(i, 1), :] = row * vals_ref[i]
        return carry
    lax.fori_loop(0, kk, loop, 0)

    xt = xt_ref[...]
    gi = jnp.dot(xt, wih_ref[...], preferred_element_type=F32) + bih_ref[...]
    gh = jnp.dot(h0_ref[...], whh_ref[...], preferred_element_type=F32) + bhh_ref[...]
    r = jax.nn.sigmoid(gi[:, :d] + gh[:, :d])
    z = jax.nn.sigmoid(gi[:, d:2 * d] + gh[:, d:2 * d])
    nn = jnp.tanh(gi[:, 2 * d:] + r * gh[:, 2 * d:])
    w_ref[...] = (1.0 - z) * nn + z * h0_ref[...]


def _gru(x, perm, vals, h0, wihT, whhT, bihr, bhhr):
    n, d = x.shape
    kk = h0.shape[0]
    grid_spec = pltpu.PrefetchScalarGridSpec(
        num_scalar_prefetch=2,
        grid=(1,),
        in_specs=[pl.BlockSpec((n, d), lambda i, p, v: (0, 0)),
                  pl.BlockSpec((kk, d), lambda i, p, v: (0, 0)),
                  pl.BlockSpec((d, 3 * d), lambda i, p, v: (0, 0)),
                  pl.BlockSpec((d, 3 * d), lambda i, p, v: (0, 0)),
                  pl.BlockSpec((1, 3 * d), lambda i, p, v: (0, 0)),
                  pl.BlockSpec((1, 3 * d), lambda i, p, v: (0, 0))],
        out_specs=pl.BlockSpec((kk, d), lambda i, p, v: (0, 0)),
        scratch_shapes=[pltpu.VMEM((kk, d), F32)],
    )
    return pl.pallas_call(
        _gru_body,
        grid_spec=grid_spec,
        out_shape=jax.ShapeDtypeStruct((kk, d), F32),
    )(perm, vals, x, h0, wihT, whhT, bihr, bhhr)


def _xw_body(x_ref, w_ref, d0_ref, d1_ref, y_ref, dinv_ref):
    dinv = lax.rsqrt(d0_ref[...] + d1_ref[...] + 1.0)
    y_ref[...] = jnp.dot(x_ref[...], w_ref[...], preferred_element_type=F32,
                         ) * dinv
    dinv_ref[...] = dinv


def _xw(x, W, d0, d1):
    n, d = x.shape
    return pl.pallas_call(
        _xw_body,
        grid=(n // _ROW_BLK,),
        in_specs=[pl.BlockSpec((_ROW_BLK, d), lambda i: (i, 0)),
                  pl.BlockSpec((d, d), lambda i: (0, 0)),
                  pl.BlockSpec((_ROW_BLK, 1), lambda i: (i, 0)),
                  pl.BlockSpec((_ROW_BLK, 1), lambda i: (i, 0))],
        out_specs=[pl.BlockSpec((_ROW_BLK, d), lambda i: (i, 0)),
                   pl.BlockSpec((_ROW_BLK, 1), lambda i: (i, 0))],
        out_shape=[jax.ShapeDtypeStruct((n, d), F32),
                   jax.ShapeDtypeStruct((n, 1), F32)],
    )(x, W, d0, d1)


def _fc2_body(x_ref, y_ref, a0_ref, a1_ref, dinv_ref, w2a_ref, w2b_ref,
              b2_ref, q_ref):
    agg = (a0_ref[...] + a1_ref[...] + y_ref[...]) * dinv_ref[...]
    q_ref[...] = (jnp.dot(x_ref[...], w2a_ref[...], preferred_element_type=F32,
                          )
                  + jnp.dot(agg, w2b_ref[...], preferred_element_type=F32,
                            )
                  + b2_ref[...])


def _fc2(x, y, a0, a1, dinv, W2a, W2b, b2r):
    n, d = x.shape
    a = W2a.shape[1]
    return pl.pallas_call(
        _fc2_body,
        grid=(n // _ROW_BLK,),
        in_specs=[pl.BlockSpec((_ROW_BLK, d), lambda i: (i, 0)),
                  pl.BlockSpec((_ROW_BLK, d), lambda i: (i, 0)),
                  pl.BlockSpec((_ROW_BLK, d), lambda i: (i, 0)),
                  pl.BlockSpec((_ROW_BLK, d), lambda i: (i, 0)),
                  pl.BlockSpec((_ROW_BLK, 1), lambda i: (i, 0)),
                  pl.BlockSpec((d, a), lambda i: (0, 0)),
                  pl.BlockSpec((d, a), lambda i: (0, 0)),
                  pl.BlockSpec((1, a), lambda i: (0, 0))],
        out_specs=pl.BlockSpec((_ROW_BLK, a), lambda i: (i, 0)),
        out_shape=jax.ShapeDtypeStruct((n, a), F32),
    )(x, y, a0, a1, dinv, W2a, W2b, b2r)


def kernel(inputs, edge_index, W1, b1, p_pool, W_ih, W_hh, b_ih, b_hh,
           init_w, W2, b2):
    n, d = inputs.shape
    e = edge_index.shape[1]
    a = W2.shape[1]

    src3 = edge_index[0].reshape(_NW, -1, _C)
    dst3 = edge_index[1].reshape(_NW, -1, _C)

    deg2 = _sc_deg(dst3, jnp.zeros((n,), F32))

    x, score = _fc1(inputs, W1, b1.reshape(1, d), p_pool.reshape(d, 1))
    vals, perm = lax.top_k(score.reshape(n), d)
    W = _gru(x, perm, vals, init_w[0], W_ih.T, W_hh.T,
             b_ih.reshape(1, 3 * d), b_hh.reshape(1, 3 * d))

    y, dinv = _xw(x, W, deg2[0].reshape(n, 1), deg2[1].reshape(n, 1))

    acc = _sc_agg(y, src3, dst3, jnp.zeros((n, d), F32))

    q = _fc2(x, y, acc[0], acc[1], dinv, W2[:d], W2[d:],
             b2.reshape(1, a))
    return q
